# Initial kernel scaffold; baseline (speedup 1.0000x reference)
import jax
import jax.numpy as jnp
from jax.experimental import pallas as pl


def kernel(op_feats, tensor_feats, edge_index, W_op, b_op, W_edge, b_edge, W_gc, b_gc, W_fin, b_fin):
    raise NotImplementedError("write your pallas kernel here")


# speedup vs baseline: 1.0819x; 1.0819x over previous
import functools

import jax
import jax.numpy as jnp
from jax.experimental import pallas as pl

N = 10000
E = 320000
NE = 5
NH = 64
EH = 8
L = 6


def _final_kernel(h_ref, w_ref, b_ref, o_ref):
    o_ref[...] = h_ref[...] @ w_ref[...] + b_ref[0]


def kernel(op_feats, tensor_feats, edge_index, W_op, b_op, W_edge, b_edge, W_gc, b_gc, W_fin, b_fin):
    h = jax.nn.elu(op_feats @ W_op + b_op)
    efs = jax.nn.elu(jnp.einsum('ed,idh->ieh', tensor_feats, W_edge) + b_edge[:, None, :])

    src = edge_index[:, 0, :].reshape(-1)
    dst = edge_index[:, 1, :]
    seg = (dst + (jnp.arange(NE, dtype=jnp.int32) * N)[:, None]).reshape(-1)

    cnt = jax.ops.segment_sum(jnp.ones((NE * E,), jnp.float32), seg,
                              num_segments=NE * N).reshape(NE, N)
    S_e = jax.ops.segment_sum(efs.reshape(NE * E, EH), seg,
                              num_segments=NE * N).reshape(NE, N, EH)
    invc = 1.0 / jnp.maximum(cnt, 1.0)
    mask = (cnt > 0).astype(jnp.float32)
    M_e = S_e * invc[..., None]

    W1 = W_gc[:, :, :NH, :]
    W2 = W_gc[:, :, NH:, :]
    Call = (jnp.einsum('ine,lieh->lnh', M_e, W2)
            + jnp.einsum('in,lih->lnh', mask, b_gc)) / NE

    for l in range(L):
        AH = jax.ops.segment_sum(h[src], seg, num_segments=NE * N).reshape(NE, N, NH)
        acc = jnp.einsum('inh,ihk->nk', AH * invc[..., None], W1[l]) / NE + Call[l]
        if l < L - 1:
            h = jax.nn.elu(h + acc)
        else:
            h = h + acc

    out = pl.pallas_call(
        _final_kernel,
        out_shape=jax.ShapeDtypeStruct((N, 1), jnp.float32),
    )(h, W_fin, b_fin)
    return out[:, 0]


# speedup vs baseline: 11.2357x; 10.3856x over previous
import functools

import jax
import jax.numpy as jnp
from jax import lax
from jax.experimental import pallas as pl
from jax.experimental.pallas import tpu as pltpu
from jax.experimental.pallas import tpu_sc as plsc

N = 10000
E = 320000
NE = 5
D_OP = 128
D_T = 16
NH = 64
EH = 8
L = 6

NPAD = 10112
NW = 32
EPW = E // NW
CK = 128
NC = 80
SLOTS = NC * CK
PADN = SLOTS - EPW
RPT = NPAD // 16

_mesh = plsc.VectorSubcoreMesh(core_axis_name="c", subcore_axis_name="s")


def _elu(x):
    return jnp.where(x > 0, x, jnp.exp(jnp.minimum(x, 0.0)) - 1.0)


def _h0_body(x_ref, w_ref, b_ref, o_ref):
    o_ref[...] = _elu(
        jnp.dot(x_ref[...], w_ref[...], preferred_element_type=jnp.float32)
        + b_ref[...][None, :])


def _efsp_body(tf_ref, w_ref, b_ref, o_ref):
    y = _elu(
        jnp.dot(tf_ref[...], w_ref[0], preferred_element_type=jnp.float32)
        + b_ref[0, 0][None, :])
    row = jnp.concatenate(
        [y, jnp.ones((EPW, 1), jnp.float32), jnp.zeros((EPW, 16 - EH - 1), jnp.float32)],
        axis=1)
    o_ref[0, 0] = jnp.concatenate(
        [row, jnp.zeros((SLOTS - EPW, 16), jnp.float32)], axis=0)


def _combine_body(last, h_ref, p_ref, invc_ref, mask_ref, me_ref,
                  w1_ref, w2_ref, b_ref, o_ref):
    h = h_ref[...]
    acc = jnp.zeros_like(h)
    for i in range(NE):
        t = (p_ref[0, i] + p_ref[1, i]) * invc_ref[:, i][:, None]
        acc = acc + jnp.dot(t, w1_ref[i], preferred_element_type=jnp.float32)
        acc = acc + jnp.dot(me_ref[i], w2_ref[i], preferred_element_type=jnp.float32)
        acc = acc + mask_ref[:, i][:, None] * b_ref[i][None, :]
    r = h + acc * (1.0 / NE)
    o_ref[...] = r if last else _elu(r)


def _final_body(h_ref, w_ref, b_ref, o_ref):
    o_ref[...] = jnp.dot(h_ref[...], w_ref[...],
                         preferred_element_type=jnp.float32) + b_ref[0]


def _zero_rows(buf, rows, width):
    z = jnp.zeros((16,), jnp.float32)

    @pl.loop(0, rows)
    def _(r):
        for k in range(width // 16):
            buf[r, pl.ds(k * 16, 16)] = z


def _pass0_body(efsp, dstw, out, tab, idx, ebuf, zbuf, sem0, sem1):
    c = lax.axis_index("c")
    s = lax.axis_index("s")
    w = c * 16 + s
    _zero_rows(zbuf, RPT, 16)
    pltpu.sync_copy(zbuf, tab.at[pl.ds(s * RPT, RPT)])
    plsc.subcore_barrier()
    for i in range(NE):
        pltpu.sync_copy(dstw.at[i, w], idx)
        pltpu.async_copy(efsp.at[i, w, pl.ds(0, CK)], ebuf.at[0], sem0)

        @pl.loop(0, NC, step=2)
        def _(ch):
            d1 = pltpu.async_copy(
                efsp.at[i, w, pl.ds((ch + 1) * CK, CK)], ebuf.at[1], sem1)
            pltpu.make_async_copy(
                efsp.at[i, w, pl.ds(ch * CK, CK)], ebuf.at[0], sem0).wait()
            pltpu.sync_copy(ebuf.at[0], tab.at[idx.at[ch]], add=True)

            @pl.when(ch + 2 < NC)
            def _():
                pltpu.async_copy(
                    efsp.at[i, w, pl.ds((ch + 2) * CK, CK)], ebuf.at[0], sem0)

            d1.wait()
            pltpu.sync_copy(ebuf.at[1], tab.at[idx.at[ch + 1]], add=True)

        plsc.subcore_barrier()
        pltpu.sync_copy(tab.at[pl.ds(s * RPT, RPT)],
                        out.at[c, i, pl.ds(s * RPT, RPT)])
        pltpu.sync_copy(zbuf, tab.at[pl.ds(s * RPT, RPT)])
        plsc.subcore_barrier()


def _layer_body(h_hbm, srcw, dstw, out, acc, sidx, didx, gbuf, zbuf, sem0, sem1):
    c = lax.axis_index("c")
    s = lax.axis_index("s")
    w = c * 16 + s
    _zero_rows(zbuf, RPT, NH)
    pltpu.sync_copy(zbuf, acc.at[pl.ds(s * RPT, RPT)])
    plsc.subcore_barrier()
    for i in range(NE):
        pltpu.sync_copy(srcw.at[i, w], sidx)
        pltpu.sync_copy(dstw.at[i, w], didx)
        pltpu.async_copy(h_hbm.at[sidx.at[0]], gbuf.at[0], sem0)

        @pl.loop(0, NC, step=2)
        def _(ch):
            d1 = pltpu.async_copy(h_hbm.at[sidx.at[ch + 1]], gbuf.at[1], sem1)
            pltpu.make_async_copy(h_hbm.at[sidx.at[ch]], gbuf.at[0], sem0).wait()
            pltpu.sync_copy(gbuf.at[0], acc.at[didx.at[ch]], add=True)

            @pl.when(ch + 2 < NC)
            def _():
                pltpu.async_copy(h_hbm.at[sidx.at[ch + 2]], gbuf.at[0], sem0)

            d1.wait()
            pltpu.sync_copy(gbuf.at[1], acc.at[didx.at[ch + 1]], add=True)

        plsc.subcore_barrier()
        pltpu.sync_copy(acc.at[pl.ds(s * RPT, RPT)],
                        out.at[c, i, pl.ds(s * RPT, RPT)])
        pltpu.sync_copy(zbuf, acc.at[pl.ds(s * RPT, RPT)])
        plsc.subcore_barrier()


_sc_params = pltpu.CompilerParams(use_tc_tiling_on_sc=False)

_pass0 = functools.partial(
    pl.kernel,
    _pass0_body,
    out_type=jax.ShapeDtypeStruct((2, NE, NPAD, 16), jnp.float32),
    mesh=_mesh,
    compiler_params=_sc_params,
    scratch_types=[
        pltpu.VMEM_SHARED((NPAD, 16), jnp.float32),
        pltpu.VMEM((NC, CK), jnp.int32),
        pltpu.VMEM((2, CK, 16), jnp.float32),
        pltpu.VMEM((RPT, 16), jnp.float32),
        pltpu.SemaphoreType.DMA,
        pltpu.SemaphoreType.DMA,
    ],
)()

_layer_agg = functools.partial(
    pl.kernel,
    _layer_body,
    out_type=jax.ShapeDtypeStruct((2, NE, NPAD, NH), jnp.float32),
    mesh=_mesh,
    compiler_params=_sc_params,
    scratch_types=[
        pltpu.VMEM_SHARED((NPAD, NH), jnp.float32),
        pltpu.VMEM((NC, CK), jnp.int32),
        pltpu.VMEM((NC, CK), jnp.int32),
        pltpu.VMEM((2, CK, NH), jnp.float32),
        pltpu.VMEM((RPT, NH), jnp.float32),
        pltpu.SemaphoreType.DMA,
        pltpu.SemaphoreType.DMA,
    ],
)()


def kernel(op_feats, tensor_feats, edge_index, W_op, b_op, W_edge, b_edge,
           W_gc, b_gc, W_fin, b_fin):
    f32 = jnp.float32

    src = edge_index[:, 0, :].reshape(NE, NW, EPW)
    dst = edge_index[:, 1, :].reshape(NE, NW, EPW)
    pad_src = jnp.broadcast_to((jnp.arange(PADN, dtype=jnp.int32) * 37) % N,
                               (NE, NW, PADN))
    pad_dst = jnp.broadcast_to(N + (jnp.arange(PADN, dtype=jnp.int32) % (NPAD - N)),
                               (NE, NW, PADN))
    srcW = jnp.concatenate([src, pad_src], axis=2).reshape(NE, NW, NC, CK)
    dstW = jnp.concatenate([dst, pad_dst], axis=2).reshape(NE, NW, NC, CK)

    opf_pad = jnp.concatenate(
        [op_feats, jnp.zeros((NPAD - N, D_OP), f32)], axis=0)
    h = pl.pallas_call(
        _h0_body,
        out_shape=jax.ShapeDtypeStruct((NPAD, NH), f32),
    )(opf_pad, W_op, b_op)

    efsp = pl.pallas_call(
        _efsp_body,
        grid=(NE, NW),
        in_specs=[
            pl.BlockSpec((EPW, D_T), lambda i, w: (w, 0)),
            pl.BlockSpec((1, D_T, EH), lambda i, w: (i, 0, 0)),
            pl.BlockSpec((1, 1, EH), lambda i, w: (i, 0, 0)),
        ],
        out_specs=pl.BlockSpec((1, 1, SLOTS, 16), lambda i, w: (i, w, 0, 0)),
        out_shape=jax.ShapeDtypeStruct((NE, NW, SLOTS, 16), f32),
    )(tensor_feats, W_edge, b_edge[:, None, :])

    p0 = _pass0(efsp, dstW)
    S = p0[0] + p0[1]
    cnt = S[..., EH]
    invc = 1.0 / jnp.maximum(cnt, 1.0)
    mask = (cnt > 0).astype(f32)
    M_e = S[..., :EH] * invc[..., None]

    W1 = W_gc[:, :, :NH, :]
    W2 = W_gc[:, :, NH:, :]

    BN = 2528
    for l in range(L):
        p = _layer_agg(h, srcW, dstW)
        h = pl.pallas_call(
            functools.partial(_combine_body, l == L - 1),
            grid=(NPAD // BN,),
            in_specs=[
                pl.BlockSpec((BN, NH), lambda r: (r, 0)),
                pl.BlockSpec((2, NE, BN, NH), lambda r: (0, 0, r, 0)),
                pl.BlockSpec((BN, NE), lambda r: (r, 0)),
                pl.BlockSpec((BN, NE), lambda r: (r, 0)),
                pl.BlockSpec((NE, BN, EH), lambda r: (0, r, 0)),
                pl.BlockSpec((NE, NH, NH), lambda r: (0, 0, 0)),
                pl.BlockSpec((NE, EH, NH), lambda r: (0, 0, 0)),
                pl.BlockSpec((NE, NH), lambda r: (0, 0)),
            ],
            out_specs=pl.BlockSpec((BN, NH), lambda r: (r, 0)),
            out_shape=jax.ShapeDtypeStruct((NPAD, NH), f32),
        )(h, p, invc.T, mask.T, M_e, W1[l], W2[l], b_gc[l])

    out = pl.pallas_call(
        _final_body,
        out_shape=jax.ShapeDtypeStruct((NPAD, 1), f32),
    )(h, W_fin, b_fin)
    return out[:N, 0]
